# baseline (speedup 1.0000x reference)
import functools

import jax
import jax.numpy as jnp
from jax import lax
from jax.experimental import pallas as pl
from jax.experimental.pallas import tpu as pltpu
from jax.experimental.pallas import tpu_sc as plsc

B, S, H, V = 4, 2048, 1024, 100000
NC, NS, L = 2, 16, 16
NW = NC * NS
SBLK = S // NW
C = 16
NCHUNK = SBLK // C
JW = H // L

_mesh = plsc.VectorSubcoreMesh(core_axis_name="c", subcore_axis_name="s")


@functools.partial(
    pl.kernel,
    mesh=_mesh,
    out_type=jax.ShapeDtypeStruct((B * S, H), jnp.float32),
    scratch_types=[
        pltpu.VMEM((SBLK, H), jnp.float32),
        pltpu.VMEM((C,), jnp.int32),
        pltpu.VMEM((C,), jnp.int32),
        pltpu.VMEM((C, H), jnp.float32),
        pltpu.VMEM((C, H), jnp.float32),
        pltpu.SemaphoreType.DMA,
        pltpu.SemaphoreType.DMA,
    ],
)
def _emb_kernel(ids_hbm, tt_hbm, w_hbm, p_hbm, t_hbm, out_hbm,
                pbuf, idv, ttv, wbuf, tbuf, sem_w, sem_t):
    wid = lax.axis_index("s") * NC + lax.axis_index("c")
    s0 = wid * SBLK
    pltpu.sync_copy(p_hbm.at[pl.ds(s0, SBLK)], pbuf)
    for b in range(B):
        for c in range(NCHUNK):
            off = b * S + s0 + c * C
            pltpu.sync_copy(ids_hbm.at[pl.ds(off, C)], idv)
            pltpu.sync_copy(tt_hbm.at[pl.ds(off, C)], ttv)
            cp_w = pltpu.async_copy(w_hbm.at[idv], wbuf, sem_w)
            cp_t = pltpu.async_copy(t_hbm.at[ttv], tbuf, sem_t)
            cp_w.wait()
            cp_t.wait()

            def row_body(r, _, c=c):
                def col_body(j, _):
                    col = pl.ds(j * L, L)
                    wbuf[r, col] = (wbuf[r, col] + tbuf[r, col]
                                    + pbuf[c * C + r, col])
                    return 0
                lax.fori_loop(0, JW, col_body, 0)
                return 0

            lax.fori_loop(0, C, row_body, 0)
            pltpu.sync_copy(wbuf, out_hbm.at[pl.ds(off, C)])


def kernel(input_ids, token_type_ids, word_embeddings, position_embeddings,
           token_type_embeddings):
    ids = input_ids.reshape(-1).astype(jnp.int32)
    tt = token_type_ids.reshape(-1).astype(jnp.int32)
    out = _emb_kernel(ids, tt, word_embeddings, position_embeddings,
                      token_type_embeddings)
    return out.reshape(B, S, H)
